# baseline (speedup 1.0000x reference)
import functools

import jax
import jax.numpy as jnp
from jax import lax
from jax.experimental import pallas as pl
from jax.experimental.pallas import tpu as pltpu
from jax.experimental.pallas import tpu_sc as plsc

BATCH = 4096
MAX_LEN = 200
EMB = 32
NC = 2
NS = 16
NW = NC * NS
BPW = BATCH // NW


def _body(idxT_hbm, table_hbm, pos_hbm, out_hbm, idx_v, buf, blk, pos_v, sem):
    wid = lax.axis_index("s") * NC + lax.axis_index("c")
    pltpu.sync_copy(idxT_hbm.at[:, pl.ds(wid * BPW, BPW)], idx_v)
    pltpu.sync_copy(pos_hbm, pos_v)

    def pos_body(l, carry):
        pltpu.async_copy(table_hbm.at[idx_v.at[l]], buf, sem).wait()
        pos_h = [pos_v[l, pl.ds(16 * h, 16)] for h in range(2)]
        f_h = [lax.iota(jnp.int32, 16) + 16 * h for h in range(2)]
        lin_h = [((f >> 3) << 10) + ((f & 7) << 7) for f in f_h]

        def tok_body(b, carry2):
            for h in range(2):
                x = buf[b, pl.ds(16 * h, 16)] + pos_h[h]
                plsc.store_scatter(blk, [lin_h[h] + b], x)
            return carry2

        lax.fori_loop(0, BPW, tok_body, 0)
        for fh in range(EMB // 8):
            pltpu.sync_copy(
                blk.at[pl.ds(fh * 8 * BPW, 8 * BPW)],
                out_hbm.at[l].at[fh].at[wid],
            )
        return carry

    lax.fori_loop(0, MAX_LEN, pos_body, 0)


@jax.jit
def _run(idxT, token_table, pos_table):
    mesh = plsc.VectorSubcoreMesh(core_axis_name="c", subcore_axis_name="s")
    k = functools.partial(
        pl.kernel,
        mesh=mesh,
        out_type=jax.ShapeDtypeStruct(
            (MAX_LEN, EMB // 8, NW, 8 * BPW), jnp.float32
        ),
        scratch_types=[
            pltpu.VMEM((MAX_LEN, BPW), jnp.int32),
            pltpu.VMEM((BPW, EMB), jnp.float32),
            pltpu.VMEM((EMB * BPW,), jnp.float32),
            pltpu.VMEM((MAX_LEN, EMB), jnp.float32),
            pltpu.SemaphoreType.DMA,
        ],
        compiler_params=pltpu.CompilerParams(
            use_tc_tiling_on_sc=False, needs_layout_passes=False
        ),
    )(_body)
    return k(idxT, token_table, pos_table)


def kernel(batch_seqs, token_table, pos_table):
    out4 = _run(batch_seqs.T, token_table, pos_table)
    out5 = out4.reshape(MAX_LEN, EMB // 8, NW, 8, BPW)
    return out5.transpose(2, 4, 0, 1, 3).reshape(BATCH, MAX_LEN, EMB)


# speedup vs baseline: 1.1853x; 1.1853x over previous
import functools

import jax
import jax.numpy as jnp
from jax import lax
from jax.experimental import pallas as pl
from jax.experimental.pallas import tpu as pltpu
from jax.experimental.pallas import tpu_sc as plsc

BATCH = 4096
MAX_LEN = 200
EMB = 32
NC = 2
NS = 16
NW = NC * NS
BPW = BATCH // NW
UNROLL = 4


def _body(idxT_hbm, table_hbm, pos_hbm, out_hbm, idx_v, buf, pos_v, sem):
    wid = lax.axis_index("s") * NC + lax.axis_index("c")
    pltpu.sync_copy(idxT_hbm.at[:, pl.ds(wid * BPW, BPW)], idx_v)
    pltpu.sync_copy(pos_hbm, pos_v)

    def pos_body(l, carry):
        pltpu.async_copy(table_hbm.at[idx_v.at[l]], buf, sem).wait()
        pos_h = [pos_v[l, pl.ds(16 * h, 16)] for h in range(2)]

        def tok_body(b4, carry2):
            for u in range(UNROLL):
                b = b4 * UNROLL + u
                for h in range(2):
                    buf[b, pl.ds(16 * h, 16)] = (
                        buf[b, pl.ds(16 * h, 16)] + pos_h[h]
                    )
            return carry2

        lax.fori_loop(0, BPW // UNROLL, tok_body, 0)
        pltpu.sync_copy(buf, out_hbm.at[l].at[wid])
        return carry

    lax.fori_loop(0, MAX_LEN, pos_body, 0)


@jax.jit
def _run(idxT, token_table, pos_table):
    mesh = plsc.VectorSubcoreMesh(core_axis_name="c", subcore_axis_name="s")
    k = functools.partial(
        pl.kernel,
        mesh=mesh,
        out_type=jax.ShapeDtypeStruct((MAX_LEN, NW, BPW, EMB), jnp.float32),
        scratch_types=[
            pltpu.VMEM((MAX_LEN, BPW), jnp.int32),
            pltpu.VMEM((BPW, EMB), jnp.float32),
            pltpu.VMEM((MAX_LEN, EMB), jnp.float32),
            pltpu.SemaphoreType.DMA,
        ],
        compiler_params=pltpu.CompilerParams(
            use_tc_tiling_on_sc=False, needs_layout_passes=False
        ),
    )(_body)
    return k(idxT, token_table, pos_table)


def kernel(batch_seqs, token_table, pos_table):
    out4 = _run(batch_seqs.T, token_table, pos_table)
    return out4.transpose(1, 2, 0, 3).reshape(BATCH, MAX_LEN, EMB)
